# Initial kernel scaffold; baseline (speedup 1.0000x reference)
import jax
import jax.numpy as jnp
from jax.experimental import pallas as pl


def kernel(x, edge_index, dis, Wl1, bl1, Wr1, Wl2, bl2, Wr2, Em1_W, Em1_b, Eu1_W, Eu1_b, Em2_W, Em2_b, Eu2_W, Eu2_b, Wfc, bfc):
    raise NotImplementedError("write your pallas kernel here")


# speedup vs baseline: 7.6486x; 7.6486x over previous
import functools

import numpy as np

import jax
import jax.numpy as jnp
from jax import lax
from jax.experimental import pallas as pl
from jax.experimental.pallas import tpu as pltpu
from jax.experimental.pallas import tpu_sc as plsc

N = 10000
E = 320000
DN = 128
NC, NS = 2, 16
NW = NC * NS
CH = 125
CPT = 80
SZ = 200
NSTR = N // SZ
BLK = 1000
GRID = N // BLK

_f32 = jnp.float32
_i32 = jnp.int32


def _leaky(v):
    return jnp.where(v >= 0, v, 0.01 * v)


def _mmT(a, b):
    return lax.dot_general(a, b, (((1,), (1,)), ((), ())),
                           preferred_element_type=_f32)


def _tc_a_body(x_ref, wl_ref, wr_ref, y1_ref, z1_ref):
    xb = x_ref[...]
    y1_ref[...] = _mmT(xb, wl_ref[...])
    z1_ref[...] = _mmT(xb, wr_ref[...])


def _tc_a(x, Wl1, Wr1):
    return pl.pallas_call(
        _tc_a_body,
        grid=(GRID,),
        in_specs=[
            pl.BlockSpec((BLK, DN), lambda i: (i, 0)),
            pl.BlockSpec((64, DN), lambda i: (0, 0)),
            pl.BlockSpec((64, DN), lambda i: (0, 0)),
        ],
        out_specs=[
            pl.BlockSpec((BLK, 64), lambda i: (i, 0)),
            pl.BlockSpec((BLK, 64), lambda i: (i, 0)),
        ],
        out_shape=[
            jax.ShapeDtypeStruct((N, 64), _f32),
            jax.ShapeDtypeStruct((N, 64), _f32),
        ],
    )(x, Wl1, Wr1)


def _tc_c_body(accY_ref, accD_ref, z1_ref, wl2_ref, wr2_ref, em1p_ref,
               eu1_ref, bl1_ref, em1b_ref, eu1b_ref,
               y2_ref, z2_ref, p_ref, cntc_ref):
    t1 = accY_ref[0] + accY_ref[1]
    dD = accD_ref[0] + accD_ref[1]
    sel4 = (lax.broadcasted_iota(_i32, (16, 1), 0) == 4).astype(_f32)
    cnt = jnp.dot(dD, sel4, preferred_element_type=_f32)
    h1 = _leaky(t1 / jnp.maximum(cnt, 1.0) + bl1_ref[...] + z1_ref[...])
    y2_ref[...] = _mmT(h1, wl2_ref[...])
    z2_ref[...] = _mmT(h1, wr2_ref[...])
    mean1e = _mmT(dD, em1p_ref[...]) / (cnt + 1.0) + em1b_ref[...]
    d1row = _leaky(_mmT(mean1e, eu1_ref[...]) + eu1b_ref[...])
    d1const = _leaky(_mmT(em1b_ref[...], eu1_ref[...]) + eu1b_ref[...])
    p_ref[...] = d1row - d1const
    cntc_ref[...] = jnp.concatenate([cnt, jnp.zeros((BLK, 15), _f32)], axis=1)


def _tc_c(accY, accD, z1, Wl2, Wr2, Em1p, Eu1_W, bl1, Em1_b, Eu1_b):
    full = lambda r, c: pl.BlockSpec((r, c), lambda i: (0, 0))
    blk = lambda c: pl.BlockSpec((BLK, c), lambda i: (i, 0))
    return pl.pallas_call(
        _tc_c_body,
        grid=(GRID,),
        in_specs=[
            pl.BlockSpec((2, BLK, 64), lambda i: (0, i, 0)),
            pl.BlockSpec((2, BLK, 16), lambda i: (0, i, 0)),
            blk(64),
            full(32, 64), full(32, 64), full(128, 16), full(64, 128),
            full(1, 64), full(1, 128), full(1, 64),
        ],
        out_specs=[blk(32), blk(32), blk(64), blk(16)],
        out_shape=[
            jax.ShapeDtypeStruct((N, 32), _f32),
            jax.ShapeDtypeStruct((N, 32), _f32),
            jax.ShapeDtypeStruct((N, 64), _f32),
            jax.ShapeDtypeStruct((N, 16), _f32),
        ],
    )(accY, accD, z1, Wl2, Wr2, Em1p, Eu1_W, bl1, Em1_b, Eu1_b)


def _tc_e_body(accT_ref, accU_ref, z2_ref, cntc_ref, em2_ref, eu2_ref,
               em2b_ref, eu2b_ref, em1b_ref, eu1_ref, eu1b_ref, bl2_ref,
               wfcx_ref, wfcd16_ref, wfcdB_ref, bfc16_ref,
               a_ref, h2_ref, r2d_ref, rc16_ref):
    t2 = accT_ref[0] + accT_ref[1]
    u = accU_ref[0] + accU_ref[1]
    sel0 = (lax.broadcasted_iota(_i32, (16, 1), 0) == 0).astype(_f32)
    cnt = jnp.dot(cntc_ref[...], sel0, preferred_element_type=_f32)
    h2 = _leaky(t2 / jnp.maximum(cnt, 1.0) + bl2_ref[...] + z2_ref[...])
    d1const = _leaky(_mmT(em1b_ref[...], eu1_ref[...]) + eu1b_ref[...])
    m2c = _mmT(d1const, em2_ref[...]) + em2b_ref[...]
    mean2 = (cnt * m2c + em2b_ref[...] + _mmT(u, em2_ref[...])) / (cnt + 1.0)
    d2row = _leaky(_mmT(mean2, eu2_ref[...]) + eu2b_ref[...])
    d2const = _leaky(_mmT(em2b_ref[...], eu2_ref[...]) + eu2b_ref[...])
    a_ref[...] = h2 * wfcx_ref[...]
    h2_ref[...] = h2
    r2d_ref[...] = _mmT(d2row - d2const, wfcd16_ref[...])
    rcrow = _mmT(d2const, wfcdB_ref[...]) + bfc16_ref[...]
    rc16_ref[...] = jnp.zeros((BLK, 16), _f32) + rcrow


def _tc_e(accT, accU, z2, cntc, Em2_W, Eu2_W, Em2_b, Eu2_b, Em1_b, Eu1_W,
          Eu1_b, bl2, Wfcx, Wfcd16, WfcdB, bfc16):
    full = lambda r, c: pl.BlockSpec((r, c), lambda i: (0, 0))
    blk = lambda c: pl.BlockSpec((BLK, c), lambda i: (i, 0))
    return pl.pallas_call(
        _tc_e_body,
        grid=(GRID,),
        in_specs=[
            pl.BlockSpec((2, BLK, 32), lambda i: (0, i, 0)),
            pl.BlockSpec((2, BLK, 64), lambda i: (0, i, 0)),
            blk(32), blk(16),
            full(64, 64), full(32, 64), full(1, 64), full(1, 32),
            full(1, 128), full(64, 128), full(1, 64), full(1, 32),
            full(1, 32), full(16, 32), full(16, 32), full(1, 16),
        ],
        out_specs=[blk(32), blk(32), blk(16), blk(16)],
        out_shape=[
            jax.ShapeDtypeStruct((N, 32), _f32),
            jax.ShapeDtypeStruct((N, 32), _f32),
            jax.ShapeDtypeStruct((N, 16), _f32),
            jax.ShapeDtypeStruct((N, 16), _f32),
        ],
    )(accT, accU, z2, cntc, Em2_W, Eu2_W, Em2_b, Eu2_b, Em1_b, Eu1_W,
      Eu1_b, bl2, Wfcx, Wfcd16, WfcdB, bfc16)


@functools.cache
def _mesh():
    return plsc.VectorSubcoreMesh(core_axis_name="c", subcore_axis_name="s",
                                  num_cores=NC, num_subcores=NS)


_SC_PARAMS = pltpu.CompilerParams(needs_layout_passes=False,
                                 use_tc_tiling_on_sc=False)


def _wid():
    return lax.axis_index("c") * NS + lax.axis_index("s")


def _sc_b_body(src2d, dst2d, dis3d, y1t, zY, zD, outY, outD,
               srcb, dstb, rows, disb, accY, accD, sem):
    c = lax.axis_index("c")
    s = lax.axis_index("s")
    wid = _wid()

    def zstripe(k, carry):
        idx = s + NS * k

        @pl.when(idx < NSTR)
        def _():
            off = pl.multiple_of(idx * SZ, 8)
            pltpu.sync_copy(zY.at[pl.ds(0, SZ)], accY.at[pl.ds(off, SZ)])
            pltpu.sync_copy(zD.at[pl.ds(0, SZ)], accD.at[pl.ds(off, SZ)])

        return carry

    lax.fori_loop(0, (NSTR + NS - 1) // NS, zstripe, 0)
    base = pl.multiple_of(wid * CPT, 8)
    pltpu.sync_copy(src2d.at[pl.ds(base, CPT)], srcb)
    pltpu.sync_copy(dst2d.at[pl.ds(base, CPT)], dstb)
    cvec = jnp.where(lax.iota(_i32, 16) == 4, 1.0, 0.0)

    def initrow(r, carry):
        disb[r] = cvec
        return carry

    lax.fori_loop(0, CH, initrow, 0)
    plsc.subcore_barrier()

    def step(j, carry):
        row = wid * CPT + j
        pltpu.sync_copy(dis3d.at[row], disb.at[:, pl.ds(0, 4)])
        pltpu.async_copy(y1t.at[srcb.at[j]], rows, sem).wait()
        pltpu.sync_copy(rows, accY.at[dstb.at[j]], add=True)
        pltpu.sync_copy(disb, accD.at[dstb.at[j]], add=True)
        return carry

    lax.fori_loop(0, CPT, step, 0)
    plsc.subcore_barrier()

    def dstripe(k, carry):
        idx = s + NS * k

        @pl.when(idx < NSTR)
        def _():
            off = pl.multiple_of(idx * SZ, 8)
            pltpu.sync_copy(accY.at[pl.ds(off, SZ)], outY.at[c, pl.ds(off, SZ)])
            pltpu.sync_copy(accD.at[pl.ds(off, SZ)], outD.at[c, pl.ds(off, SZ)])

        return carry

    lax.fori_loop(0, (NSTR + NS - 1) // NS, dstripe, 0)


@functools.cache
def _sc_b_kernel():
    return pl.kernel(
        _sc_b_body,
        mesh=_mesh(),
        compiler_params=_SC_PARAMS,
        out_type=(
            jax.ShapeDtypeStruct((NC, N, 64), _f32),
            jax.ShapeDtypeStruct((NC, N, 16), _f32),
        ),
        scratch_types=[
            pltpu.VMEM((CPT, CH), _i32),
            pltpu.VMEM((CPT, CH), _i32),
            pltpu.VMEM((CH, 64), _f32),
            pltpu.VMEM((CH, 16), _f32),
            pltpu.VMEM_SHARED((N, 64), _f32),
            pltpu.VMEM_SHARED((N, 16), _f32),
            pltpu.SemaphoreType.DMA,
        ],
    )


def _sc_d_body(src2d, dst2d, dstu3, y2t, po3, zY, outT, outU,
               srcb, dstb, rows, dstu, rowsU, accT, accU, sem):
    c = lax.axis_index("c")
    s = lax.axis_index("s")
    wid = _wid()

    def zstripe(k, carry):
        idx = s + NS * k

        @pl.when(idx < NSTR)
        def _():
            off = pl.multiple_of(idx * SZ, 8)
            pltpu.sync_copy(zY.at[pl.ds(0, SZ), pl.ds(0, 32)],
                            accT.at[pl.ds(off, SZ)])
            pltpu.sync_copy(zY.at[pl.ds(0, SZ)], accU.at[pl.ds(off, SZ)])

        return carry

    lax.fori_loop(0, (NSTR + NS - 1) // NS, zstripe, 0)
    base = pl.multiple_of(wid * CPT, 8)
    pltpu.sync_copy(src2d.at[pl.ds(base, CPT)], srcb)
    pltpu.sync_copy(dst2d.at[pl.ds(base, CPT)], dstb)
    plsc.subcore_barrier()

    def step(j, carry):
        pltpu.async_copy(y2t.at[srcb.at[j]], rows, sem).wait()
        pltpu.sync_copy(rows, accT.at[dstb.at[j]], add=True)
        return carry

    lax.fori_loop(0, CPT, step, 0)

    def ustep(k, carry):
        cu = wid + NW * k

        @pl.when(cu < CPT)
        def _():
            pltpu.sync_copy(dstu3.at[cu], dstu)
            pltpu.sync_copy(po3.at[cu], rowsU)
            pltpu.sync_copy(rowsU, accU.at[dstu.at[0]], add=True)

        return carry

    lax.fori_loop(0, (CPT + NW - 1) // NW, ustep, 0)
    plsc.subcore_barrier()

    def dstripe(k, carry):
        idx = s + NS * k

        @pl.when(idx < NSTR)
        def _():
            off = pl.multiple_of(idx * SZ, 8)
            pltpu.sync_copy(accT.at[pl.ds(off, SZ)], outT.at[c, pl.ds(off, SZ)])
            pltpu.sync_copy(accU.at[pl.ds(off, SZ)], outU.at[c, pl.ds(off, SZ)])

        return carry

    lax.fori_loop(0, (NSTR + NS - 1) // NS, dstripe, 0)


@functools.cache
def _sc_d_kernel():
    return pl.kernel(
        _sc_d_body,
        mesh=_mesh(),
        compiler_params=_SC_PARAMS,
        out_type=(
            jax.ShapeDtypeStruct((NC, N, 32), _f32),
            jax.ShapeDtypeStruct((NC, N, 64), _f32),
        ),
        scratch_types=[
            pltpu.VMEM((CPT, CH), _i32),
            pltpu.VMEM((CPT, CH), _i32),
            pltpu.VMEM((CH, 32), _f32),
            pltpu.VMEM((1, CH), _i32),
            pltpu.VMEM((CH, 64), _f32),
            pltpu.VMEM_SHARED((N, 32), _f32),
            pltpu.VMEM_SHARED((N, 64), _f32),
            pltpu.SemaphoreType.DMA,
        ],
    )


def _sc_f_body(src2d, dst2d, at_, h2t, r3d, rc16, outO,
               srcb, dstb, bufA, bufH, rbuf, rcv, obuf, sem):
    wid = _wid()
    base = pl.multiple_of(wid * CPT, 8)
    pltpu.sync_copy(src2d.at[pl.ds(base, CPT)], srcb)
    pltpu.sync_copy(dst2d.at[pl.ds(base, CPT)], dstb)
    pltpu.sync_copy(rc16.at[pl.ds(0, 1)], rcv)
    rcvec = rcv[0]
    zero16 = jnp.zeros((16,), _f32)

    def zrow(r, carry):
        rbuf[r] = zero16
        return carry

    lax.fori_loop(0, 128, zrow, 0)
    zidx = jnp.zeros((16,), _i32)

    def step(j, carry):
        row = wid * CPT + j
        ga = pltpu.async_copy(at_.at[srcb.at[j]], bufA.at[pl.ds(0, CH)], sem)
        gh = pltpu.async_copy(h2t.at[dstb.at[j]], bufH.at[pl.ds(0, CH)], sem)

        @pl.when(wid == 0)
        def _():
            pltpu.sync_copy(r3d.at[j], rbuf.at[pl.ds(0, CH)])

        ga.wait()
        gh.wait()
        for g in range(8):
            ridx = lax.iota(_i32, 16) + g * 16
            acc = rcvec + plsc.load_gather(rbuf, [ridx, zidx])
            for k in range(32):
                kidx = jnp.full((16,), k, _i32)
                va = plsc.load_gather(bufA, [ridx, kidx])
                vh = plsc.load_gather(bufH, [ridx, kidx])
                acc = acc + va * vh
            if g < 7:
                obuf[0, pl.ds(g * 16, 16)] = acc
            else:
                plsc.store_scatter(obuf, [zidx, ridx], acc,
                                   mask=lax.iota(_i32, 16) < CH - 112)
        pltpu.sync_copy(obuf, outO.at[row])
        return carry

    lax.fori_loop(0, CPT, step, 0)


@functools.cache
def _sc_f_kernel():
    return pl.kernel(
        _sc_f_body,
        mesh=_mesh(),
        compiler_params=_SC_PARAMS,
        out_type=jax.ShapeDtypeStruct((E // CH, 1, CH), _f32),
        scratch_types=[
            pltpu.VMEM((CPT, CH), _i32),
            pltpu.VMEM((CPT, CH), _i32),
            pltpu.VMEM((128, 32), _f32),
            pltpu.VMEM((128, 32), _f32),
            pltpu.VMEM((128, 16), _f32),
            pltpu.VMEM((1, 16), _f32),
            pltpu.VMEM((1, CH), _f32),
            pltpu.SemaphoreType.DMA,
        ],
    )


def kernel(x, edge_index, dis, Wl1, bl1, Wr1, Wl2, bl2, Wr2,
           Em1_W, Em1_b, Eu1_W, Eu1_b, Em2_W, Em2_b, Eu2_W, Eu2_b, Wfc, bfc):
    src2d = edge_index[0].reshape(E // CH, CH)
    dst2d = edge_index[1].reshape(E // CH, CH)
    dstu3 = dst2d[:CPT].reshape(CPT, 1, CH)
    dis3d = dis.reshape(E // CH, CH, 4)
    zY = jnp.zeros((SZ, 64), _f32)
    zD = jnp.zeros((SZ, 16), _f32)
    Em1p = jnp.pad(Em1_W, ((0, 0), (0, 12)))
    r1 = lambda b: b.reshape(1, -1)

    y1, z1 = _tc_a(x, Wl1, Wr1)
    accY, accD = _sc_b_kernel()(src2d, dst2d, dis3d, y1, zY, zD)
    y2t, z2, po, cntc = _tc_c(accY, accD, z1, Wl2, Wr2, Em1p, Eu1_W,
                              r1(bl1), r1(Em1_b), r1(Eu1_b))
    po3 = po.reshape(CPT, CH, 64)
    accT, accU = _sc_d_kernel()(src2d, dst2d, dstu3, y2t, po3, zY)
    wfc_d = Wfc[:, 32:]
    Wfcd16 = jnp.concatenate([wfc_d, jnp.zeros((15, 32), _f32)], axis=0)
    WfcdB = jnp.broadcast_to(wfc_d, (16, 32))
    bfc16 = jnp.broadcast_to(bfc.reshape(1, 1), (1, 16))
    at_, h2t, r2d, rc16 = _tc_e(accT, accU, z2, cntc, Em2_W, Eu2_W,
                                r1(Em2_b), r1(Eu2_b), r1(Em1_b), Eu1_W,
                                r1(Eu1_b), r1(bl2), Wfc[:, :32], Wfcd16,
                                WfcdB, bfc16)
    r3d = r2d.reshape(CPT, CH, 16)
    outO = _sc_f_kernel()(src2d, dst2d, at_, h2t, r3d, rc16)
    return outO.reshape(E, 1)
